# Initial kernel scaffold; baseline (speedup 1.0000x reference)
import jax
import jax.numpy as jnp
from jax.experimental import pallas as pl


def kernel(x, edge_index, W1, b1, gamma, beta, W2, b2, Wfc, bfc):
    raise NotImplementedError("write your pallas kernel here")


# speedup vs baseline: 14.9233x; 14.9233x over previous
import functools

import jax
import jax.numpy as jnp
from jax import lax
from jax.experimental import pallas as pl
from jax.experimental.pallas import tpu as pltpu
from jax.experimental.pallas import tpu_sc as plsc

N = 10000
E = 320000
D = 128

NC = 2
NS = 16
NW = NC * NS

C = 128
NCHUNK = E // C
TILE_CHUNKS = -(-NCHUNK // NW)

NP = 10240
HL = 16

_mesh = plsc.VectorSubcoreMesh(core_axis_name="c", subcore_axis_name="s")


def _sc_hist(edge_index):
  @functools.partial(
      pl.kernel,
      mesh=_mesh,
      out_type=jax.ShapeDtypeStruct((NC, NP, HL), jnp.float32),
      scratch_types=[
          pltpu.VMEM((C,), jnp.int32),
          pltpu.VMEM((C, HL), jnp.float32),
          pltpu.VMEM_SHARED((NP, HL), jnp.float32),
      ],
  )
  def hist_kernel(ei_hbm, out_hbm, idx_v, ones_v, hist_sh):
    c = lax.axis_index("c")
    s = lax.axis_index("s")
    w = c * NS + s

    @pl.loop(0, C)
    def _(r):
      ones_v[r, pl.ds(0, HL)] = jnp.zeros((HL,), jnp.float32)

    rows_per_tile = NP // NS

    @pl.loop(0, rows_per_tile // C)
    def _(i):
      pltpu.sync_copy(ones_v, hist_sh.at[pl.ds(s * rows_per_tile + i * C, C)])

    @pl.loop(0, C)
    def _(r):
      ones_v[r, pl.ds(0, HL)] = jnp.full((HL,), 1.0, jnp.float32)

    plsc.subcore_barrier()

    @pl.loop(0, TILE_CHUNKS)
    def _(k):
      g = k * NW + w

      @pl.when(g < NCHUNK)
      def _():
        pltpu.sync_copy(ei_hbm.at[1, pl.ds(g * C, C)], idx_v)
        pltpu.sync_copy(ones_v, hist_sh.at[idx_v], add=True)

    plsc.subcore_barrier()

    pltpu.sync_copy(hist_sh.at[pl.ds(s * rows_per_tile, rows_per_tile)],
                    out_hbm.at[c, pl.ds(s * rows_per_tile, rows_per_tile)])

  return hist_kernel(edge_index)


def _sc_scatter(hp, edge_index):
  @functools.partial(
      pl.kernel,
      mesh=_mesh,
      out_type=jax.ShapeDtypeStruct((NC, NP, D), jnp.float32),
      scratch_types=[
          pltpu.VMEM((C,), jnp.int32),
          pltpu.VMEM((C,), jnp.int32),
          pltpu.VMEM((C, D), jnp.float32),
          pltpu.SemaphoreType.DMA,
          pltpu.VMEM_SHARED((NP, D), jnp.float32),
      ],
  )
  def scat_kernel(hp_hbm, ei_hbm, out_hbm, src_v, dst_v, rows_v, sem, acc_sh):
    c = lax.axis_index("c")
    s = lax.axis_index("s")
    w = c * NS + s

    @pl.loop(0, C)
    def _(r):
      @pl.loop(0, D, step=16)
      def _(j):
        rows_v[r, pl.ds(j, 16)] = jnp.zeros((16,), jnp.float32)

    rows_per_tile = NP // NS

    @pl.loop(0, rows_per_tile // C)
    def _(i):
      pltpu.sync_copy(rows_v,
                      acc_sh.at[pl.ds(s * rows_per_tile + i * C, C)])

    plsc.subcore_barrier()

    @pl.loop(0, TILE_CHUNKS)
    def _(k):
      g = k * NW + w

      @pl.when(g < NCHUNK)
      def _():
        pltpu.sync_copy(ei_hbm.at[0, pl.ds(g * C, C)], src_v)
        pltpu.sync_copy(ei_hbm.at[1, pl.ds(g * C, C)], dst_v)
        pltpu.async_copy(hp_hbm.at[src_v], rows_v, sem).wait()
        pltpu.sync_copy(rows_v, acc_sh.at[dst_v], add=True)

    plsc.subcore_barrier()

    pltpu.sync_copy(acc_sh.at[pl.ds(s * rows_per_tile, rows_per_tile)],
                    out_hbm.at[c, pl.ds(s * rows_per_tile, rows_per_tile)])

  return scat_kernel(hp, edge_index)


def _tc_mm(x, w):
  def body(x_ref, w_ref, o_ref):
    o_ref[...] = jnp.dot(x_ref[...], w_ref[...],
                         preferred_element_type=jnp.float32)

  return pl.pallas_call(
      body, out_shape=jax.ShapeDtypeStruct((x.shape[0], w.shape[1]),
                                           jnp.float32))(x, w)


def _tc_scale(h1, hist_t):
  def body(h_ref, hist_ref, hp_ref, dinv_ref):
    deg = hist_ref[:, 0:1] + hist_ref[:, 1:2] + 1.0
    dinv = lax.rsqrt(deg)
    dinv_ref[...] = dinv
    hp_ref[...] = h_ref[...] * dinv

  return pl.pallas_call(
      body,
      out_shape=[
          jax.ShapeDtypeStruct((N, D), jnp.float32),
          jax.ShapeDtypeStruct((N, 1), jnp.float32),
      ])(h1, hist_t)


def _tc_mid(s1, h1p, dinv, b1, gamma, beta, w2):
  def body(s_ref, h1p_ref, dinv_ref, b1_ref, g_ref, be_ref, w2_ref, o_ref):
    ssum = (s_ref[0] + s_ref[1])[:N]
    h = jax.nn.relu((ssum + h1p_ref[...]) * dinv_ref[...] + b1_ref[...])
    mean = jnp.mean(h, axis=0, keepdims=True)
    var = jnp.mean((h - mean) ** 2, axis=0, keepdims=True)
    hbn = (h - mean) * lax.rsqrt(var + 1e-5) * g_ref[...] + be_ref[...]
    h2 = jnp.dot(hbn, w2_ref[...], preferred_element_type=jnp.float32)
    o_ref[...] = h2 * dinv_ref[...]

  return pl.pallas_call(
      body, out_shape=jax.ShapeDtypeStruct((N, D), jnp.float32))(
          s1, h1p, dinv, b1, gamma, beta, w2)


def _tc_final(s2, h2p, dinv, b2, wfc, bfc):
  def body(s_ref, h2p_ref, dinv_ref, b2_ref, wfc_ref, bfc_ref, o_ref):
    ssum = (s_ref[0] + s_ref[1])[:N]
    h = jax.nn.relu((ssum + h2p_ref[...]) * dinv_ref[...] + b2_ref[...])
    o = jnp.dot(h, wfc_ref[...], preferred_element_type=jnp.float32)
    o_ref[...] = jax.nn.sigmoid(o + bfc_ref[...])

  return pl.pallas_call(
      body, out_shape=jax.ShapeDtypeStruct((N, 1), jnp.float32))(
          s2, h2p, dinv, b2, wfc, bfc)


def kernel(x, edge_index, W1, b1, gamma, beta, W2, b2, Wfc, bfc):
  hist = _sc_hist(edge_index)
  h1 = _tc_mm(x, W1)
  hist_t = hist[:, :N, 0].T
  h1p, dinv = _tc_scale(h1, hist_t)
  s1 = _sc_scatter(h1p, edge_index)
  h2p = _tc_mid(s1, h1p, dinv, b1.reshape(1, D), gamma.reshape(1, D),
                beta.reshape(1, D), W2)
  s2 = _sc_scatter(h2p, edge_index)
  return _tc_final(s2, h2p, dinv, b2.reshape(1, D), Wfc, bfc.reshape(1, 1))


# speedup vs baseline: 23.9443x; 1.6045x over previous
import functools

import jax
import jax.numpy as jnp
from jax import lax
from jax.experimental import pallas as pl
from jax.experimental.pallas import tpu as pltpu
from jax.experimental.pallas import tpu_sc as plsc

N = 10000
E = 320000
D = 128

NC = 2
NS = 16
NW = NC * NS

C = 80
NCHUNK = E // C
KPT = NCHUNK // NW

NP = 10240
HL = 16

_mesh = plsc.VectorSubcoreMesh(core_axis_name="c", subcore_axis_name="s")


def _sc_hist(edge_index):
  HC = 128
  HCHUNK = E // HC
  TILE_CHUNKS = -(-HCHUNK // NW)

  @functools.partial(
      pl.kernel,
      mesh=_mesh,
      out_type=jax.ShapeDtypeStruct((NC, NP, HL), jnp.float32),
      scratch_types=[
          pltpu.VMEM((HC,), jnp.int32),
          pltpu.VMEM((HC, HL), jnp.float32),
          pltpu.VMEM_SHARED((NP, HL), jnp.float32),
      ],
  )
  def hist_kernel(ei_hbm, out_hbm, idx_v, ones_v, hist_sh):
    c = lax.axis_index("c")
    s = lax.axis_index("s")
    w = c * NS + s

    @pl.loop(0, HC)
    def _(r):
      ones_v[r, pl.ds(0, HL)] = jnp.zeros((HL,), jnp.float32)

    rpt = NP // NS

    @pl.loop(0, rpt // HC)
    def _(i):
      pltpu.sync_copy(ones_v, hist_sh.at[pl.ds(s * rpt + i * HC, HC)])

    @pl.loop(0, HC)
    def _(r):
      ones_v[r, pl.ds(0, HL)] = jnp.full((HL,), 1.0, jnp.float32)

    plsc.subcore_barrier()

    @pl.loop(0, TILE_CHUNKS)
    def _(k):
      g = k * NW + w

      @pl.when(g < HCHUNK)
      def _():
        pltpu.sync_copy(ei_hbm.at[1, pl.ds(g * HC, HC)], idx_v)
        pltpu.sync_copy(ones_v, hist_sh.at[idx_v], add=True)

    plsc.subcore_barrier()

    pltpu.sync_copy(hist_sh.at[pl.ds(s * rpt, rpt)],
                    out_hbm.at[c, pl.ds(s * rpt, rpt)])

  return hist_kernel(edge_index)


def _sc_scatter(hp, ei2, ei3):
  @functools.partial(
      pl.kernel,
      mesh=_mesh,
      out_type=jax.ShapeDtypeStruct((NC, NP, D), jnp.float32),
      scratch_types=[
          pltpu.VMEM((KPT * C,), jnp.int32),
          pltpu.VMEM((KPT, C), jnp.int32),
          pltpu.VMEM((C, D), jnp.float32),
          pltpu.VMEM((C, D), jnp.float32),
          pltpu.SemaphoreType.DMA,
          pltpu.SemaphoreType.DMA,
          pltpu.VMEM_SHARED((NP, D), jnp.float32),
      ],
  )
  def scat_kernel(hp_hbm, ei2_hbm, ei_hbm, out_hbm, srcs_v, dsts_v, rows_a,
                  rows_b, sem_a, sem_b, acc_sh):
    c = lax.axis_index("c")
    s = lax.axis_index("s")
    w = c * NS + s

    pltpu.sync_copy(ei2_hbm.at[0, w], srcs_v)
    pltpu.sync_copy(ei_hbm.at[1, w], dsts_v)

    @pl.loop(0, C)
    def _(r):
      @pl.loop(0, D, step=16)
      def _(j):
        rows_a[r, pl.ds(j, 16)] = jnp.zeros((16,), jnp.float32)

    rpt = NP // NS

    @pl.loop(0, rpt // C)
    def _(i):
      pltpu.sync_copy(rows_a, acc_sh.at[pl.ds(s * rpt + i * C, C)])

    plsc.subcore_barrier()

    def _gather(k, buf, sem):
      return pltpu.async_copy(hp_hbm.at[srcs_v.at[pl.ds(k * C, C)]], buf, sem)

    def _gather_wait(k, buf, sem):
      pltpu.make_async_copy(hp_hbm.at[srcs_v.at[pl.ds(k * C, C)]], buf,
                            sem).wait()

    _gather(0, rows_a, sem_a)

    @pl.loop(0, KPT - 1, step=2)
    def _(k):
      _gather(k + 1, rows_b, sem_b)
      _gather_wait(k, rows_a, sem_a)
      pltpu.sync_copy(rows_a, acc_sh.at[dsts_v.at[k]], add=True)
      _gather(k + 2, rows_a, sem_a)
      _gather_wait(k + 1, rows_b, sem_b)
      pltpu.sync_copy(rows_b, acc_sh.at[dsts_v.at[k + 1]], add=True)

    _gather_wait(KPT - 1, rows_a, sem_a)
    pltpu.sync_copy(rows_a, acc_sh.at[dsts_v.at[KPT - 1]], add=True)

    plsc.subcore_barrier()

    pltpu.sync_copy(acc_sh.at[pl.ds(s * rpt, rpt)],
                    out_hbm.at[c, pl.ds(s * rpt, rpt)])

  return scat_kernel(hp, ei2, ei3)


def _tc_mm(x, w):
  def body(x_ref, w_ref, o_ref):
    o_ref[...] = jnp.dot(x_ref[...], w_ref[...],
                         preferred_element_type=jnp.float32)

  return pl.pallas_call(
      body, out_shape=jax.ShapeDtypeStruct((x.shape[0], w.shape[1]),
                                           jnp.float32))(x, w)


def _tc_scale(h1, hist_t):
  def body(h_ref, hist_ref, hp_ref, dinv_ref):
    deg = hist_ref[:, 0:1] + hist_ref[:, 1:2] + 1.0
    dinv = lax.rsqrt(deg)
    dinv_ref[...] = dinv
    hp_ref[...] = h_ref[...] * dinv

  return pl.pallas_call(
      body,
      out_shape=[
          jax.ShapeDtypeStruct((N, D), jnp.float32),
          jax.ShapeDtypeStruct((N, 1), jnp.float32),
      ])(h1, hist_t)


def _tc_mid(s1, h1p, dinv, b1, gamma, beta, w2):
  def body(s_ref, h1p_ref, dinv_ref, b1_ref, g_ref, be_ref, w2_ref, o_ref):
    ssum = (s_ref[0] + s_ref[1])[:N]
    h = jax.nn.relu((ssum + h1p_ref[...]) * dinv_ref[...] + b1_ref[...])
    mean = jnp.mean(h, axis=0, keepdims=True)
    var = jnp.mean((h - mean) ** 2, axis=0, keepdims=True)
    hbn = (h - mean) * lax.rsqrt(var + 1e-5) * g_ref[...] + be_ref[...]
    h2 = jnp.dot(hbn, w2_ref[...], preferred_element_type=jnp.float32)
    o_ref[...] = h2 * dinv_ref[...]

  return pl.pallas_call(
      body, out_shape=jax.ShapeDtypeStruct((N, D), jnp.float32))(
          s1, h1p, dinv, b1, gamma, beta, w2)


def _tc_final(s2, h2p, dinv, b2, wfc, bfc):
  def body(s_ref, h2p_ref, dinv_ref, b2_ref, wfc_ref, bfc_ref, o_ref):
    ssum = (s_ref[0] + s_ref[1])[:N]
    h = jax.nn.relu((ssum + h2p_ref[...]) * dinv_ref[...] + b2_ref[...])
    o = jnp.dot(h, wfc_ref[...], preferred_element_type=jnp.float32)
    o_ref[...] = jax.nn.sigmoid(o + bfc_ref[...])

  return pl.pallas_call(
      body, out_shape=jax.ShapeDtypeStruct((N, 1), jnp.float32))(
          s2, h2p, dinv, b2, wfc, bfc)


def kernel(x, edge_index, W1, b1, gamma, beta, W2, b2, Wfc, bfc):
  ei3 = edge_index.reshape(2, NW, KPT, C)
  ei2 = edge_index.reshape(2, NW, KPT * C)
  hist = _sc_hist(edge_index)
  h1 = _tc_mm(x, W1)
  hist_t = hist[:, :N, 0].T
  h1p, dinv = _tc_scale(h1, hist_t)
  s1 = _sc_scatter(h1p, ei2, ei3)
  h2p = _tc_mid(s1, h1p, dinv, b1.reshape(1, D), gamma.reshape(1, D),
                beta.reshape(1, D), W2)
  s2 = _sc_scatter(h2p, ei2, ei3)
  return _tc_final(s2, h2p, dinv, b2.reshape(1, D), Wfc, bfc.reshape(1, 1))


# speedup vs baseline: 24.9082x; 1.0403x over previous
import functools

import jax
import jax.numpy as jnp
from jax import lax
from jax.experimental import pallas as pl
from jax.experimental.pallas import tpu as pltpu
from jax.experimental.pallas import tpu_sc as plsc

N = 10000
E = 320000
D = 128

NC = 2
NS = 16
NW = NC * NS

NP = 10240
EP = NW * NP
C = 128
KPT = NP // C
EPT = NP

HL = 16

_mesh = plsc.VectorSubcoreMesh(core_axis_name="c", subcore_axis_name="s")


def _sc_hist(edge_index):
  HCHUNK = E // C
  TILE_CHUNKS = -(-HCHUNK // NW)

  @functools.partial(
      pl.kernel,
      mesh=_mesh,
      out_type=jax.ShapeDtypeStruct((NC, NP, HL), jnp.float32),
      scratch_types=[
          pltpu.VMEM((C,), jnp.int32),
          pltpu.VMEM((C, HL), jnp.float32),
          pltpu.VMEM_SHARED((NP, HL), jnp.float32),
      ],
  )
  def hist_kernel(ei_hbm, out_hbm, idx_v, ones_v, hist_sh):
    c = lax.axis_index("c")
    s = lax.axis_index("s")
    w = c * NS + s

    @pl.loop(0, C)
    def _(r):
      ones_v[r, pl.ds(0, HL)] = jnp.zeros((HL,), jnp.float32)

    rpt = NP // NS

    @pl.loop(0, rpt // C)
    def _(i):
      pltpu.sync_copy(ones_v, hist_sh.at[pl.ds(s * rpt + i * C, C)])

    @pl.loop(0, C)
    def _(r):
      ones_v[r, pl.ds(0, HL)] = jnp.full((HL,), 1.0, jnp.float32)

    plsc.subcore_barrier()

    @pl.loop(0, TILE_CHUNKS)
    def _(k):
      g = k * NW + w

      @pl.when(g < HCHUNK)
      def _():
        pltpu.sync_copy(ei_hbm.at[1, pl.ds(g * C, C)], idx_v)
        pltpu.sync_copy(ones_v, hist_sh.at[idx_v], add=True)

    plsc.subcore_barrier()

    pltpu.sync_copy(hist_sh.at[pl.ds(s * rpt, rpt)],
                    out_hbm.at[c, pl.ds(s * rpt, rpt)])

  return hist_kernel(edge_index)


def _sc_scatter(hp, ei2):
  @functools.partial(
      pl.kernel,
      mesh=_mesh,
      out_type=jax.ShapeDtypeStruct((NC, NP, D), jnp.float32),
      scratch_types=[
          pltpu.VMEM((EPT,), jnp.int32),
          pltpu.VMEM((C,), jnp.int32),
          pltpu.VMEM((C,), jnp.int32),
          pltpu.VMEM((C, D), jnp.float32),
          pltpu.VMEM((C, D), jnp.float32),
          pltpu.SemaphoreType.DMA,
          pltpu.SemaphoreType.DMA,
          pltpu.VMEM_SHARED((NP, D), jnp.float32),
      ],
  )
  def scat_kernel(hp_hbm, ei2_hbm, out_hbm, srcs_v, dst_a, dst_b, rows_a,
                  rows_b, sem_a, sem_b, acc_sh):
    c = lax.axis_index("c")
    s = lax.axis_index("s")
    w = c * NS + s

    pltpu.sync_copy(ei2_hbm.at[0, w], srcs_v)

    @pl.loop(0, C)
    def _(r):
      @pl.loop(0, D, step=16)
      def _(j):
        rows_a[r, pl.ds(j, 16)] = jnp.zeros((16,), jnp.float32)

    rpt = NP // NS

    @pl.loop(0, rpt // C)
    def _(i):
      pltpu.sync_copy(rows_a, acc_sh.at[pl.ds(s * rpt + i * C, C)])

    plsc.subcore_barrier()

    def _gather(k, buf, sem):
      return pltpu.async_copy(hp_hbm.at[srcs_v.at[pl.ds(k * C, C)]], buf, sem)

    def _gather_wait(k, buf, sem):
      pltpu.make_async_copy(hp_hbm.at[srcs_v.at[pl.ds(k * C, C)]], buf,
                            sem).wait()

    def _dst_load(k, dbuf):
      pltpu.sync_copy(ei2_hbm.at[1, w, pl.ds(k * C, C)], dbuf)

    def _scatter(buf, dbuf):
      pltpu.sync_copy(buf, acc_sh.at[dbuf], add=True)

    _gather(0, rows_a, sem_a)
    _dst_load(0, dst_a)

    @pl.loop(0, KPT - 2, step=2)
    def _(k):
      _gather(k + 1, rows_b, sem_b)
      _dst_load(k + 1, dst_b)
      _gather_wait(k, rows_a, sem_a)
      _scatter(rows_a, dst_a)
      _gather(k + 2, rows_a, sem_a)
      _dst_load(k + 2, dst_a)
      _gather_wait(k + 1, rows_b, sem_b)
      _scatter(rows_b, dst_b)

    _gather(KPT - 1, rows_b, sem_b)
    _dst_load(KPT - 1, dst_b)
    _gather_wait(KPT - 2, rows_a, sem_a)
    _scatter(rows_a, dst_a)
    _gather_wait(KPT - 1, rows_b, sem_b)
    _scatter(rows_b, dst_b)

    plsc.subcore_barrier()

    pltpu.sync_copy(acc_sh.at[pl.ds(s * rpt, rpt)],
                    out_hbm.at[c, pl.ds(s * rpt, rpt)])

  return scat_kernel(hp, ei2)


def _tc_mm(x, w):
  def body(x_ref, w_ref, o_ref):
    o_ref[...] = jnp.dot(x_ref[...], w_ref[...],
                         preferred_element_type=jnp.float32)

  return pl.pallas_call(
      body, out_shape=jax.ShapeDtypeStruct((x.shape[0], w.shape[1]),
                                           jnp.float32))(x, w)


def _tc_scale(h1, hist_t):
  def body(h_ref, hist_ref, hp_ref, dinv_ref):
    deg = hist_ref[:, 0:1] + hist_ref[:, 1:2] + 1.0
    dinv = lax.rsqrt(deg)
    dinv_ref[...] = dinv
    hp_ref[...] = h_ref[...] * dinv

  return pl.pallas_call(
      body,
      out_shape=[
          jax.ShapeDtypeStruct((N, D), jnp.float32),
          jax.ShapeDtypeStruct((N, 1), jnp.float32),
      ])(h1, hist_t)


def _tc_mid(s1, h1p, dinv, b1, gamma, beta, w2):
  def body(s_ref, h1p_ref, dinv_ref, b1_ref, g_ref, be_ref, w2_ref, o_ref):
    ssum = (s_ref[0] + s_ref[1])[:N]
    h = jax.nn.relu((ssum + h1p_ref[...]) * dinv_ref[...] + b1_ref[...])
    mean = jnp.mean(h, axis=0, keepdims=True)
    var = jnp.mean((h - mean) ** 2, axis=0, keepdims=True)
    hbn = (h - mean) * lax.rsqrt(var + 1e-5) * g_ref[...] + be_ref[...]
    h2 = jnp.dot(hbn, w2_ref[...], preferred_element_type=jnp.float32)
    o_ref[...] = h2 * dinv_ref[...]

  return pl.pallas_call(
      body, out_shape=jax.ShapeDtypeStruct((N, D), jnp.float32))(
          s1, h1p, dinv, b1, gamma, beta, w2)


def _tc_final(s2, h2p, dinv, b2, wfc, bfc):
  def body(s_ref, h2p_ref, dinv_ref, b2_ref, wfc_ref, bfc_ref, o_ref):
    ssum = (s_ref[0] + s_ref[1])[:N]
    h = jax.nn.relu((ssum + h2p_ref[...]) * dinv_ref[...] + b2_ref[...])
    o = jnp.dot(h, wfc_ref[...], preferred_element_type=jnp.float32)
    o_ref[...] = jax.nn.sigmoid(o + bfc_ref[...])

  return pl.pallas_call(
      body, out_shape=jax.ShapeDtypeStruct((N, 1), jnp.float32))(
          s2, h2p, dinv, b2, wfc, bfc)


def kernel(x, edge_index, W1, b1, gamma, beta, W2, b2, Wfc, bfc):
  npad = EP - E
  pad_src = (jnp.arange(npad, dtype=jnp.int32) * 131) % N
  pad_dst = N + (jnp.arange(npad, dtype=jnp.int32) % (NP - N))
  ei_p = jnp.concatenate([edge_index, jnp.stack([pad_src, pad_dst])], axis=1)
  ei2 = ei_p.reshape(2, NW, EPT)

  hist = _sc_hist(edge_index)
  h1 = _tc_mm(x, W1)
  hist_t = hist[:, :N, 0].T
  h1p, dinv = _tc_scale(h1, hist_t)
  s1 = _sc_scatter(h1p, ei2)
  h2p = _tc_mid(s1, h1p, dinv, b1.reshape(1, D), gamma.reshape(1, D),
                beta.reshape(1, D), W2)
  s2 = _sc_scatter(h2p, ei2)
  return _tc_final(s2, h2p, dinv, b2.reshape(1, D), Wfc, bfc.reshape(1, 1))
